# Initial kernel scaffold; baseline (speedup 1.0000x reference)
import jax
import jax.numpy as jnp
from jax.experimental import pallas as pl


def kernel(h, edge_index, W_fc, attn_l, attn_r, gat_bias, W_ih, W_hh, b_ih, b_hh):
    raise NotImplementedError("write your pallas kernel here")


# speedup vs baseline: 15.5653x; 15.5653x over previous
import functools

import jax
import jax.numpy as jnp
from jax import lax
from jax.experimental import pallas as pl
from jax.experimental.pallas import tpu as pltpu
from jax.experimental.pallas import tpu_sc as plsc

N = 10000
E = 320000
F = 128
H = 8
HF = H * F

NC = 2
NS = 16
L = 16

HEADS_PER_CORE = H // NC
EDGES_PER_TILE = E // NS
CHUNK = 80
NCHUNK = EDGES_PER_TILE // CHUNK
NPAD = 10240
DEN_PER_TILE = NPAD // NS
RST_PER_TILE = NPAD // NS
ZROWS = 16
SUPER = 25

BN = 400


def _phase1_body(h_ref, wfc_ref, al_ref, ar_ref,
                 feat_ref, el_ref, er_ref, elmax_ref, ermax_ref):
    hb = h_ref[...]
    feat = lax.dot_general(hb, wfc_ref[...], (((1,), (1,)), ((), ())),
                           preferred_element_type=jnp.float32)
    feat_ref[...] = feat
    el = jnp.dot(feat, al_ref[...], preferred_element_type=jnp.float32)
    er = jnp.dot(feat, ar_ref[...], preferred_element_type=jnp.float32)
    el_ref[...] = el
    er_ref[...] = er
    cur_el = jnp.broadcast_to(jnp.max(el, axis=0, keepdims=True), (8, H))
    cur_er = jnp.broadcast_to(jnp.max(er, axis=0, keepdims=True), (8, H))

    @pl.when(pl.program_id(0) == 0)
    def _():
        elmax_ref[...] = cur_el
        ermax_ref[...] = cur_er

    @pl.when(pl.program_id(0) > 0)
    def _():
        elmax_ref[...] = jnp.maximum(elmax_ref[...], cur_el)
        ermax_ref[...] = jnp.maximum(ermax_ref[...], cur_er)


def _phase1(h, W_fc, A_l, A_r):
    grid = (N // BN,)
    return pl.pallas_call(
        _phase1_body,
        grid=grid,
        in_specs=[
            pl.BlockSpec((BN, F), lambda i: (i, 0)),
            pl.BlockSpec((HF, F), lambda i: (0, 0)),
            pl.BlockSpec((HF, H), lambda i: (0, 0)),
            pl.BlockSpec((HF, H), lambda i: (0, 0)),
        ],
        out_specs=[
            pl.BlockSpec((BN, HF), lambda i: (i, 0)),
            pl.BlockSpec((BN, H), lambda i: (i, 0)),
            pl.BlockSpec((BN, H), lambda i: (i, 0)),
            pl.BlockSpec((8, H), lambda i: (0, 0)),
            pl.BlockSpec((8, H), lambda i: (0, 0)),
        ],
        out_shape=[
            jax.ShapeDtypeStruct((N, HF), jnp.float32),
            jax.ShapeDtypeStruct((N, H), jnp.float32),
            jax.ShapeDtypeStruct((N, H), jnp.float32),
            jax.ShapeDtypeStruct((8, H), jnp.float32),
            jax.ShapeDtypeStruct((8, H), jnp.float32),
        ],
    )(h, W_fc, A_l, A_r)


def _sc_body(feat_hbm, elT_hbm, erT_hbm, src_hbm, dst_hbm, m_hbm, zeros_hbm,
             rst_hbm, den_hbm,
             rst_sp,
             el_v, er_v, den_v, src_s, dst_s,
             idxf_v, dstc_v, w_v, rows_v, zero_v, m_v,
             sem):
    cid = lax.axis_index("c")
    sid = lax.axis_index("s")
    ebase = sid * EDGES_PER_TILE

    pltpu.sync_copy(m_hbm, m_v)
    pltpu.sync_copy(zeros_hbm, zero_v)

    def head_body(hh, carry):
        head = cid * HEADS_PER_CORE + hh

        def zden(i, c):
            den_v[pl.ds(i * L, L)] = jnp.zeros((L,), jnp.float32)
            return c
        lax.fori_loop(0, NPAD // L, zden, 0)
        for k in range(RST_PER_TILE // ZROWS):
            pltpu.sync_copy(
                zero_v, rst_sp.at[pl.ds(sid * RST_PER_TILE + k * ZROWS, ZROWS)])

        pltpu.sync_copy(elT_hbm.at[head], el_v)
        pltpu.sync_copy(erT_hbm.at[head], er_v)

        plsc.subcore_barrier()

        m_h = plsc.load_gather(m_v, [jnp.full((L,), head, jnp.int32)])

        def super_body(sp, c1):
            sbase = ebase + sp * SUPER * CHUNK
            pltpu.sync_copy(src_hbm.at[pl.ds(sbase, SUPER * CHUNK)], src_s)
            pltpu.sync_copy(dst_hbm.at[pl.ds(sbase, SUPER * CHUNK)], dst_s)

            def chunk_body(c, c2):
                base = c * CHUNK
                for g in range(CHUNK // L):
                    s16 = src_s[pl.ds(base + g * L, L)]
                    d16 = dst_s[pl.ds(base + g * L, L)]
                    idxf_v[pl.ds(g * L, L)] = s16 * H + head
                    dstc_v[pl.ds(g * L, L)] = d16
                cp = pltpu.async_copy(feat_hbm.at[idxf_v], rows_v, sem)
                for g in range(CHUNK // L):
                    s16 = src_s[pl.ds(base + g * L, L)]
                    d16 = dst_s[pl.ds(base + g * L, L)]
                    z = (plsc.load_gather(el_v, [s16])
                         + plsc.load_gather(er_v, [d16]))
                    z = jnp.where(z > 0, z, 0.2 * z)
                    w16 = jnp.exp(z - m_h)
                    w_v[pl.ds(g * L, L)] = w16
                    plsc.addupdate_scatter(den_v, [d16], w16)
                cp.wait()

                def scale(j, c3):
                    wj = plsc.load_gather(w_v, [jnp.full((L,), j, jnp.int32)])
                    for q in range(F // L):
                        rows_v[j, pl.ds(q * L, L)] = (
                            rows_v[j, pl.ds(q * L, L)] * wj)
                    return c3
                lax.fori_loop(0, CHUNK, scale, 0)

                pltpu.sync_copy(rows_v, rst_sp.at[dstc_v], add=True)
                return c2

            lax.fori_loop(0, SUPER, chunk_body, 0)
            return c1

        lax.fori_loop(0, NCHUNK // SUPER, super_body, 0)

        plsc.subcore_barrier()

        pltpu.sync_copy(
            rst_sp.at[pl.ds(sid * RST_PER_TILE, RST_PER_TILE)],
            rst_hbm.at[head, pl.ds(sid * RST_PER_TILE, RST_PER_TILE)])
        pltpu.sync_copy(den_v, den_hbm.at[head, sid])
        return carry

    lax.fori_loop(0, HEADS_PER_CORE, head_body, 0)


def _phase2(feat_rows, elT, erT, src, dst, m16, zeros):
    mesh = plsc.VectorSubcoreMesh(core_axis_name="c", subcore_axis_name="s")
    fn = pl.kernel(
        _sc_body,
        out_type=[
            jax.ShapeDtypeStruct((H, NPAD, F), jnp.float32),
            jax.ShapeDtypeStruct((H, NS, NPAD), jnp.float32),
        ],
        mesh=mesh,
        compiler_params=pltpu.CompilerParams(needs_layout_passes=False),
        scratch_types=[
            pltpu.VMEM_SHARED((NPAD, F), jnp.float32),
            pltpu.VMEM((NPAD,), jnp.float32),
            pltpu.VMEM((NPAD,), jnp.float32),
            pltpu.VMEM((NPAD,), jnp.float32),
            pltpu.VMEM((SUPER * CHUNK,), jnp.int32),
            pltpu.VMEM((SUPER * CHUNK,), jnp.int32),
            pltpu.VMEM((CHUNK,), jnp.int32),
            pltpu.VMEM((CHUNK,), jnp.int32),
            pltpu.VMEM((CHUNK,), jnp.float32),
            pltpu.VMEM((CHUNK, F), jnp.float32),
            pltpu.VMEM((ZROWS, F), jnp.float32),
            pltpu.VMEM((L,), jnp.float32),
            pltpu.SemaphoreType.DMA,
        ],
    )
    return fn(feat_rows, elT, erT, src, dst, m16, zeros)


def _phase3_body(rst_ref, den_ref, h_ref, wih_ref, whh_ref,
                 bih_ref, bhh_ref, gb_ref, out_ref):
    hb = h_ref[...]
    den = den_ref[...]
    den = jnp.where(den == 0.0, 1.0, den)
    gi = jnp.zeros((BN, 3 * F), jnp.float32)
    for hh in range(H):
        x_h = (rst_ref[hh] / den[:, hh:hh + 1]
               + gb_ref[:, hh * F:(hh + 1) * F])
        gi = gi + lax.dot_general(
            x_h, wih_ref[:, hh * F:(hh + 1) * F],
            (((1,), (1,)), ((), ())), preferred_element_type=jnp.float32)
    gi = gi + bih_ref[...]
    gh = lax.dot_general(hb, whh_ref[...], (((1,), (1,)), ((), ())),
                         preferred_element_type=jnp.float32) + bhh_ref[...]
    i_r, i_z, i_n = gi[:, :F], gi[:, F:2 * F], gi[:, 2 * F:]
    h_r, h_z, h_n = gh[:, :F], gh[:, F:2 * F], gh[:, 2 * F:]
    r = jax.nn.sigmoid(i_r + h_r)
    z = jax.nn.sigmoid(i_z + h_z)
    ntil = jnp.tanh(i_n + r * h_n)
    h_new = (1.0 - z) * ntil + z * hb
    out_ref[...] = jnp.where(h_new > 0, h_new, jnp.exp(h_new) - 1.0)


def _phase3(rst, denT, h, W_ih, W_hh, b_ih2, b_hh2, gb2):
    grid = (N // BN,)
    return pl.pallas_call(
        _phase3_body,
        grid=grid,
        in_specs=[
            pl.BlockSpec((H, BN, F), lambda i: (0, i, 0)),
            pl.BlockSpec((BN, H), lambda i: (i, 0)),
            pl.BlockSpec((BN, F), lambda i: (i, 0)),
            pl.BlockSpec((3 * F, HF), lambda i: (0, 0)),
            pl.BlockSpec((3 * F, F), lambda i: (0, 0)),
            pl.BlockSpec((1, 3 * F), lambda i: (0, 0)),
            pl.BlockSpec((1, 3 * F), lambda i: (0, 0)),
            pl.BlockSpec((1, HF), lambda i: (0, 0)),
        ],
        out_specs=pl.BlockSpec((BN, F), lambda i: (i, 0)),
        out_shape=jax.ShapeDtypeStruct((N, F), jnp.float32),
    )(rst, denT, h, W_ih, W_hh, b_ih2, b_hh2, gb2)


def kernel(h, edge_index, W_fc, attn_l, attn_r, gat_bias, W_ih, W_hh, b_ih, b_hh):
    h = h.astype(jnp.float32)
    src = edge_index[0].astype(jnp.int32)
    dst = edge_index[1].astype(jnp.int32)

    eye = jnp.eye(H, dtype=jnp.float32)
    A_l = (attn_l[:, :, None] * eye[:, None, :]).reshape(HF, H)
    A_r = (attn_r[:, :, None] * eye[:, None, :]).reshape(HF, H)

    feat, el, er, elmax, ermax = _phase1(h, W_fc, A_l, A_r)

    m8 = elmax[0] + ermax[0]
    m8 = jnp.where(m8 > 0, m8, 0.2 * m8)
    m16 = jnp.concatenate([m8, jnp.zeros((L - H,), jnp.float32)])

    elT = jnp.pad(el.T, ((0, 0), (0, NPAD - N)))
    erT = jnp.pad(er.T, ((0, 0), (0, NPAD - N)))
    feat_rows = feat.reshape(N * H, F)
    zeros = jnp.zeros((ZROWS, F), jnp.float32)

    rst, den = _phase2(feat_rows, elT, erT, src, dst, m16, zeros)

    denT = den.sum(axis=1)[:, :N].T
    b_ih2 = b_ih.reshape(1, 3 * F)
    b_hh2 = b_hh.reshape(1, 3 * F)
    gb2 = gat_bias.reshape(1, HF)

    return _phase3(rst, denT, h, W_ih, W_hh, b_ih2, b_hh2, gb2)
